# baseline (speedup 1.0000x reference)
import functools

import jax
import jax.numpy as jnp
from jax import lax
from jax.experimental import pallas as pl
from jax.experimental.pallas import tpu as pltpu
from jax.experimental.pallas import tpu_sc as plsc

VOCAB = 100000
HIDDEN = 128
B, S = 1024, 200
N = B * S
NW = 32
SEQ_PER_W = N // (NW * S)
NPAIR = SEQ_PER_W // 2
UNROLL = 4

_GSLICES = ((0, 104), (104, 96), (200, 104), (304, 96))


def _body(ids_hbm, tok_hbm, pos_hbm, out_hbm,
          idx0, idx1, slab, pos_v,
          sem_g, sem_i0, sem_i1, sem_s0, sem_s1):
  nc = 2
  wid = lax.axis_index("s") * nc + lax.axis_index("c")
  base0 = wid * (SEQ_PER_W * S)

  idx_refs = [idx0, idx1]
  sem_i = [sem_i0, sem_i1]
  sem_s = [sem_s0, sem_s1]

  pltpu.sync_copy(pos_hbm.at[pl.ds(0, S)], pos_v)

  def icp(m, p):
    return pltpu.make_async_copy(
        ids_hbm.at[pl.ds(base0 + m * 2 * S, 2 * S)], idx_refs[p], sem_i[p])

  def gcps(p):
    return [
        pltpu.make_async_copy(
            tok_hbm.at[idx_refs[p].at[pl.ds(o, n)]],
            slab.at[pl.ds(p * 2 * S + o, n)], sem_g)
        for o, n in _GSLICES
    ]

  def scp_part(m, p, off, n):
    return pltpu.make_async_copy(
        slab.at[pl.ds(p * 2 * S + off, n)],
        out_hbm.at[pl.ds(base0 + m * 2 * S + off, n)], sem_s[p])

  _SPARTS = ((0, 104), (S, 104), (104, 96), (S + 104, 96))

  def swait_pair(m, p):
    for off, n in _SPARTS:
      scp_part(m, p, off, n).wait()

  def add_rows(p, lo, hi):
    r_base = p * 2 * S

    def per_iter(i, _):
      r0 = i * UNROLL
      for rr in range(UNROLL):
        r = r0 + rr
        for k in range(HIDDEN // 16):
          sl = pl.ds(k * 16, 16)
          v = pos_v[r, sl]
          plsc.addupdate(slab.at[r_base + r, sl], v)
          plsc.addupdate(slab.at[r_base + S + r, sl], v)
      return ()

    lax.fori_loop(lo // UNROLL, hi // UNROLL, per_iter, (), unroll=False)

  def add_scatter_pair(m, p):
    add_rows(p, 0, 104)
    scp_part(m, p, 0, 104).start()
    scp_part(m, p, S, 104).start()
    add_rows(p, 104, 200)
    scp_part(m, p, 104, 96).start()
    scp_part(m, p, S + 104, 96).start()

  def pstep(m, p, do_swait, do_prev, do_inext=True):
    icp(m, p).wait()
    if do_swait:
      swait_pair(m - 2, p)
    for cp in gcps(p):
      cp.start()
    if do_prev:
      q = 1 - p
      for cp in gcps(q):
        cp.wait()
      if do_inext:
        icp(m + 1, q).start()
      add_scatter_pair(m - 1, q)

  icp(0, 0).start()
  icp(1, 1).start()
  pstep(0, 0, False, False)
  pstep(1, 1, False, True)

  def round_body(t, _):
    pstep(2 * t, 0, True, True)
    pstep(2 * t + 1, 1, True, True)
    return ()

  lax.fori_loop(1, NPAIR // 2 - 1, round_body, (), unroll=False)

  pstep(14, 0, True, True)
  pstep(15, 1, True, True, do_inext=False)
  for cp in gcps(1):
    cp.wait()
  add_scatter_pair(15, 1)
  swait_pair(14, 0)
  swait_pair(15, 1)


@jax.jit
def kernel(input_ids, token_table, pos_table):
  ids_flat = input_ids.reshape(N)
  mesh = plsc.VectorSubcoreMesh(core_axis_name="c", subcore_axis_name="s")
  run = functools.partial(
      pl.kernel,
      mesh=mesh,
      out_type=jax.ShapeDtypeStruct((N, HIDDEN), jnp.float32),
      scratch_types=[
          pltpu.VMEM((2 * S,), jnp.int32),
          pltpu.VMEM((2 * S,), jnp.int32),
          pltpu.VMEM((4 * S, HIDDEN), jnp.float32),
          pltpu.VMEM((S, HIDDEN), jnp.float32),
      ] + [pltpu.SemaphoreType.DMA] * 5,
  )(_body)
  out = run(ids_flat, token_table, pos_table)
  return out.reshape(B, S, HIDDEN)


# speedup vs baseline: 1.0039x; 1.0039x over previous
import functools

import jax
import jax.numpy as jnp
from jax import lax
from jax.experimental import pallas as pl
from jax.experimental.pallas import tpu as pltpu
from jax.experimental.pallas import tpu_sc as plsc

VOCAB = 100000
HIDDEN = 128
B, S = 1024, 200
N = B * S
NW = 32
SEQ_PER_W = N // (NW * S)
NPAIR = SEQ_PER_W // 2
UNROLL = 4

_GSLICES = ((0, 104), (104, 96), (200, 104), (304, 96))


def _body(ids_hbm, tok_hbm, pos_hbm, out_hbm,
          idx0, idx1, slab, pos_v,
          sem_g, sem_i0, sem_i1, sem_s0, sem_s1):
  nc = 2
  wid = lax.axis_index("s") * nc + lax.axis_index("c")
  base0 = wid * (SEQ_PER_W * S)

  idx_refs = [idx0, idx1]
  sem_i = [sem_i0, sem_i1]
  sem_s = [sem_s0, sem_s1]

  pltpu.sync_copy(pos_hbm.at[pl.ds(0, S)], pos_v)

  def icp(m, p):
    return pltpu.make_async_copy(
        ids_hbm.at[pl.ds(base0 + m * 2 * S, 2 * S)], idx_refs[p], sem_i[p])

  def gcps(p):
    return [
        pltpu.make_async_copy(
            tok_hbm.at[idx_refs[p].at[pl.ds(o, n)]],
            slab.at[pl.ds(p * 2 * S + o, n)], sem_g)
        for o, n in _GSLICES
    ]

  def scp_part(m, p, off, n):
    return pltpu.make_async_copy(
        slab.at[pl.ds(p * 2 * S + off, n)],
        out_hbm.at[pl.ds(base0 + m * 2 * S + off, n)], sem_s[p])

  _SPARTS = ((0, 104), (S, 104), (104, 96), (S + 104, 96))

  def swait_pair(m, p):
    pltpu.make_async_copy(
        out_hbm.at[pl.ds(base0 + m * 2 * S, 2 * S)],
        slab.at[pl.ds(p * 2 * S, 2 * S)], sem_s[p]).wait()

  def add_rows(p, lo, hi):
    r_base = p * 2 * S

    def per_iter(i, _):
      r0 = i * UNROLL
      for rr in range(UNROLL):
        r = r0 + rr
        for k in range(HIDDEN // 16):
          sl = pl.ds(k * 16, 16)
          v = pos_v[r, sl]
          plsc.addupdate(slab.at[r_base + r, sl], v)
          plsc.addupdate(slab.at[r_base + S + r, sl], v)
      return ()

    lax.fori_loop(lo // UNROLL, hi // UNROLL, per_iter, (), unroll=False)

  def add_scatter_pair(m, p):
    add_rows(p, 0, 104)
    scp_part(m, p, 0, 104).start()
    scp_part(m, p, S, 104).start()
    add_rows(p, 104, 200)
    scp_part(m, p, 104, 96).start()
    scp_part(m, p, S + 104, 96).start()

  def pstep(m, p, do_swait, do_prev, do_inext=True):
    icp(m, p).wait()
    if do_swait:
      swait_pair(m - 2, p)
    for cp in gcps(p):
      cp.start()
    if do_prev:
      q = 1 - p
      for cp in gcps(q):
        cp.wait()
      if do_inext:
        icp(m + 1, q).start()
      add_scatter_pair(m - 1, q)

  icp(0, 0).start()
  icp(1, 1).start()
  pstep(0, 0, False, False)
  pstep(1, 1, False, True)

  def round_body(t, _):
    pstep(2 * t, 0, True, True)
    pstep(2 * t + 1, 1, True, True)
    return ()

  lax.fori_loop(1, NPAIR // 2 - 1, round_body, (), unroll=False)

  pstep(14, 0, True, True)
  pstep(15, 1, True, True, do_inext=False)
  for cp in gcps(1):
    cp.wait()
  add_scatter_pair(15, 1)
  swait_pair(14, 0)
  swait_pair(15, 1)


@jax.jit
def kernel(input_ids, token_table, pos_table):
  ids_flat = input_ids.reshape(N)
  mesh = plsc.VectorSubcoreMesh(core_axis_name="c", subcore_axis_name="s")
  run = functools.partial(
      pl.kernel,
      mesh=mesh,
      out_type=jax.ShapeDtypeStruct((N, HIDDEN), jnp.float32),
      scratch_types=[
          pltpu.VMEM((2 * S,), jnp.int32),
          pltpu.VMEM((2 * S,), jnp.int32),
          pltpu.VMEM((4 * S, HIDDEN), jnp.float32),
          pltpu.VMEM((S, HIDDEN), jnp.float32),
      ] + [pltpu.SemaphoreType.DMA] * 5,
  )(_body)
  out = run(ids_flat, token_table, pos_table)
  return out.reshape(B, S, HIDDEN)


# speedup vs baseline: 1.0060x; 1.0021x over previous
import functools

import jax
import jax.numpy as jnp
from jax import lax
from jax.experimental import pallas as pl
from jax.experimental.pallas import tpu as pltpu
from jax.experimental.pallas import tpu_sc as plsc

VOCAB = 100000
HIDDEN = 128
B, S = 1024, 200
N = B * S
NW = 32
SEQ_PER_W = N // (NW * S)
NPAIR = SEQ_PER_W // 2
UNROLL = 4

_GSLICES = ((0, 104), (104, 96), (200, 104), (304, 96))


def _body(ids_hbm, tok_hbm, pos_hbm, out_hbm,
          idx0, idx1, slab, pos_v,
          sem_g, sem_i0, sem_i1, sem_s0, sem_s1):
  nc = 2
  wid = lax.axis_index("s") * nc + lax.axis_index("c")
  base0 = wid * (SEQ_PER_W * S)

  idx_refs = [idx0, idx1]
  sem_i = [sem_i0, sem_i1]
  sem_s = [sem_s0, sem_s1]

  pltpu.sync_copy(pos_hbm.at[pl.ds(0, S)], pos_v)

  def icp(m, p):
    return pltpu.make_async_copy(
        ids_hbm.at[pl.ds(base0 + m * 2 * S, 2 * S)], idx_refs[p], sem_i[p])

  def gcps(p):
    return [
        pltpu.make_async_copy(
            tok_hbm.at[idx_refs[p].at[pl.ds(o, n)]],
            slab.at[pl.ds(p * 2 * S + o, n)], sem_g)
        for o, n in _GSLICES
    ]

  def scp(m, p):
    return pltpu.make_async_copy(
        slab.at[pl.ds(p * 2 * S, 2 * S)],
        out_hbm.at[pl.ds(base0 + m * 2 * S, 2 * S)], sem_s[p])

  def add_pair(p):
    r_base = p * 2 * S

    def per_iter(i, _):
      r0 = i * UNROLL
      for rr in range(UNROLL):
        r = r0 + rr
        for k in range(HIDDEN // 16):
          sl = pl.ds(k * 16, 16)
          v = pos_v[r, sl]
          plsc.addupdate(slab.at[r_base + r, sl], v)
          plsc.addupdate(slab.at[r_base + S + r, sl], v)
      return ()

    lax.fori_loop(0, S // UNROLL, per_iter, (), unroll=False)

  def pstep(m, p, do_swait, do_prev, do_inext=True):
    icp(m, p).wait()
    if do_swait:
      scp(m - 2, p).wait()
    for cp in gcps(p):
      cp.start()
    if do_prev:
      q = 1 - p
      for cp in gcps(q):
        cp.wait()
      if do_inext:
        icp(m + 1, q).start()
      add_pair(q)
      scp(m - 1, q).start()

  icp(0, 0).start()
  icp(1, 1).start()
  pstep(0, 0, False, False)
  pstep(1, 1, False, True)

  def round_body(t, _):
    pstep(2 * t, 0, True, True)
    pstep(2 * t + 1, 1, True, True)
    return ()

  lax.fori_loop(1, NPAIR // 2 - 1, round_body, (), unroll=False)

  pstep(14, 0, True, True)
  pstep(15, 1, True, True, do_inext=False)
  for cp in gcps(1):
    cp.wait()
  add_pair(1)
  scp(15, 1).start()
  scp(14, 0).wait()
  scp(15, 1).wait()


@jax.jit
def kernel(input_ids, token_table, pos_table):
  ids_flat = input_ids.reshape(N)
  mesh = plsc.VectorSubcoreMesh(core_axis_name="c", subcore_axis_name="s")
  run = functools.partial(
      pl.kernel,
      mesh=mesh,
      out_type=jax.ShapeDtypeStruct((N, HIDDEN), jnp.float32),
      scratch_types=[
          pltpu.VMEM((2 * S,), jnp.int32),
          pltpu.VMEM((2 * S,), jnp.int32),
          pltpu.VMEM((4 * S, HIDDEN), jnp.float32),
          pltpu.VMEM((S, HIDDEN), jnp.float32),
      ] + [pltpu.SemaphoreType.DMA] * 5,
  )(_body)
  out = run(ids_flat, token_table, pos_table)
  return out.reshape(B, S, HIDDEN)
